# baseline (speedup 1.0000x reference)
import functools

import jax
import jax.numpy as jnp
from jax import lax
from jax.experimental import pallas as pl
from jax.experimental.pallas import tpu as pltpu
from jax.experimental.pallas import tpu_sc as plsc

B = 16384
NF = 13
V = 100000
Vp = 102400
D = 32
DIN = NF * D
H1, H2 = 128, 64
EPS_BN = 1e-5
EPS_NORM = 1e-12

VB = 4096
NVB = Vp // VB
PR = NF * Vp


def _pack_body(src, dst):
    x = src[0]
    dst[:, 0:D] = jnp.swapaxes(x, 0, 1).astype(jnp.bfloat16)


def _pack_call(tabT):
    return pl.pallas_call(
        _pack_body,
        grid=(NF, NVB),
        in_specs=[pl.BlockSpec((1, D, VB), lambda f, vc: (f, 0, vc))],
        out_specs=pl.BlockSpec((VB, 128), lambda f, vc: (f * NVB + vc, 0)),
        out_shape=jax.ShapeDtypeStruct((PR, 128), jnp.bfloat16),
    )(tabT)


NC, NS = 2, 16
NW = NC * NS
RW = B // NW
IPW = RW * NF
IPG = 128
G = IPW // IPG
NBUF = 4
NGRP = G // NBUF


def _sc_gather_body(tab, rows_hbm, out, rows_v, pad_bufs, sem_g, sem_o):
    wid = lax.axis_index("s") * NC + lax.axis_index("c")
    pltpu.sync_copy(rows_hbm.at[wid], rows_v)
    obase = wid * IPW

    def group(g, carry):
        gets = [
            pltpu.async_copy(tab.at[rows_v.at[g * NBUF + b]],
                             pad_bufs.at[b], sem_g)
            for b in range(NBUF)
        ]
        puts = []
        for b in range(NBUF):
            gets[b].wait()
            puts.append(pltpu.async_copy(
                pad_bufs.at[b, :, pl.ds(0, D)],
                out.at[pl.ds(obase + (g * NBUF + b) * IPG, IPG)],
                sem_o))
        for put in puts:
            put.wait()
        return carry

    lax.fori_loop(0, NGRP, group, 0)


@functools.cache
def _sc_gather_call():
    mesh = plsc.VectorSubcoreMesh(core_axis_name="c", subcore_axis_name="s",
                                  num_cores=NC, num_subcores=NS)
    return pl.kernel(
        _sc_gather_body,
        mesh=mesh,
        out_type=jax.ShapeDtypeStruct((B * NF, D), jnp.bfloat16),
        scratch_types=[pltpu.VMEM((G, IPG), jnp.int32),
                       pltpu.VMEM((NBUF, IPG, 128), jnp.bfloat16),
                       pltpu.SemaphoreType.DMA,
                       pltpu.SemaphoreType.DMA],
        compiler_params=pltpu.CompilerParams(use_tc_tiling_on_sc=False,
                                             needs_layout_passes=False),
    )


BLK = 1024
NB = B // BLK


def _mlp_body(uf, itf, uW1r, iW1r, uW2r, iW2r, uv1, iv1, uv2, iv2, out,
              h1u, h1i, h2u, h2i, s1u, s1i, s2u, s2i):
    p = pl.program_id(0)
    i = pl.program_id(1)
    towers = ((uf, uW1r, uW2r, uv1, uv2, h1u, h2u, s1u, s2u),
              (itf, iW1r, iW2r, iv1, iv2, h1i, h2i, s1i, s2i))

    @pl.when(p == 0)
    def _():
        for f_ref, W1r, _W2r, _v1, _v2, h1, _h2, s1, _s2 in towers:
            x = f_ref[...]
            h = jnp.dot(x, W1r[...], preferred_element_type=jnp.float32)
            h1[pl.ds(i * BLK, BLK), :] = h
            st = jnp.concatenate(
                [jnp.sum(h, axis=0, keepdims=True),
                 jnp.sum(h * h, axis=0, keepdims=True)], axis=0)

            @pl.when(i == 0)
            def _():
                s1[0:2, :] = st

            @pl.when(i > 0)
            def _():
                s1[0:2, :] = s1[0:2, :] + st

    @pl.when(p == 1)
    def _():
        for _f, _W1r, W2r, v1, _v2, h1, h2, s1, s2 in towers:
            mu = s1[0:1, :] * (1.0 / B)
            var = s1[1:2, :] * (1.0 / B) - mu * mu
            a = v1[0:1, :] * lax.rsqrt(var + EPS_BN)
            c = v1[1:2, :] - mu * a
            h = h1[pl.ds(i * BLK, BLK), :]
            t = jnp.tanh(h * a + c)
            h2blk = jnp.dot(t, W2r[...], preferred_element_type=jnp.float32)
            h2[pl.ds(i * BLK, BLK), :] = h2blk
            st = jnp.concatenate(
                [jnp.sum(h2blk, axis=0, keepdims=True),
                 jnp.sum(h2blk * h2blk, axis=0, keepdims=True)], axis=0)

            @pl.when(i == 0)
            def _():
                s2[0:2, :] = st

            @pl.when(i > 0)
            def _():
                s2[0:2, :] = s2[0:2, :] + st

    @pl.when(p == 2)
    def _():
        zs = []
        for _f, _W1r, _W2r, _v1, v2, _h1, h2, _s1, s2 in towers:
            mu = s2[0:1, :] * (1.0 / B)
            var = s2[1:2, :] * (1.0 / B) - mu * mu
            a = v2[0:1, :] * lax.rsqrt(var + EPS_BN)
            c = v2[1:2, :] - mu * a
            zs.append(jnp.tanh(h2[pl.ds(i * BLK, BLK), :] * a + c))
        zu, zi = zs
        nu = jnp.maximum(jnp.sqrt(jnp.sum(zu * zu, axis=1, keepdims=True)),
                         EPS_NORM)
        ni = jnp.maximum(jnp.sqrt(jnp.sum(zi * zi, axis=1, keepdims=True)),
                         EPS_NORM)
        score = jnp.sum(zu * zi, axis=1, keepdims=True) / (nu * ni)
        out[pl.ds(i * BLK, BLK), :] = score


def _mlp_call(u_feat, i_feat, uW1, iW1, uW2, iW2, uv1, iv1, uv2, iv2):
    feat_spec = pl.BlockSpec((BLK, DIN),
                             lambda p, i: (jnp.where(p == 0, i, NB - 1), 0))
    whole = lambda shape: pl.BlockSpec(shape, lambda p, i: (0, 0))
    return pl.pallas_call(
        _mlp_body,
        grid=(3, NB),
        in_specs=[feat_spec, feat_spec,
                  whole((DIN, H1)), whole((DIN, H1)),
                  whole((H1, H2)), whole((H1, H2)),
                  whole((8, H1)), whole((8, H1)),
                  whole((8, H2)), whole((8, H2))],
        out_specs=pl.BlockSpec((B, 1), lambda p, i: (0, 0)),
        out_shape=jax.ShapeDtypeStruct((B, 1), jnp.float32),
        scratch_shapes=[pltpu.VMEM((B, H1), jnp.float32),
                        pltpu.VMEM((B, H1), jnp.float32),
                        pltpu.VMEM((B, H2), jnp.float32),
                        pltpu.VMEM((B, H2), jnp.float32),
                        pltpu.VMEM((8, H1), jnp.float32),
                        pltpu.VMEM((8, H1), jnp.float32),
                        pltpu.VMEM((8, H2), jnp.float32),
                        pltpu.VMEM((8, H2), jnp.float32)],
    )(u_feat, i_feat, uW1, iW1, uW2, iW2, uv1, iv1, uv2, iv2)


def _pack_bn(g, be):
    v = jnp.stack([g, be])
    return jnp.concatenate([v, jnp.zeros((6, v.shape[1]), jnp.float32)], axis=0)


def kernel(user_inputs, item_inputs, user_tables, item_tables,
           uW1, ub1, ug1, ube1, uW2, ub2, ug2, ube2,
           iW1, ib1, ig1, ibe1, iW2, ib2, ig2, ibe2):
    offs = (jnp.arange(NF, dtype=jnp.int32) * Vp)[None, :]
    u_rows = (user_inputs.astype(jnp.int32) + offs).reshape(NW, G, IPG)
    i_rows = (item_inputs.astype(jnp.int32) + offs).reshape(NW, G, IPG)
    tabT_u = jnp.swapaxes(user_tables, 1, 2)
    tabT_i = jnp.swapaxes(item_tables, 1, 2)
    pu = _pack_call(tabT_u)
    u_flat = _sc_gather_call()(pu, u_rows)
    pi = _pack_call(tabT_i)
    i_flat = _sc_gather_call()(pi, i_rows)
    u_feat = u_flat.reshape(B, DIN)
    i_feat = i_flat.reshape(B, DIN)
    score = _mlp_call(u_feat, i_feat, uW1.astype(jnp.bfloat16),
                      iW1.astype(jnp.bfloat16), uW2, iW2,
                      _pack_bn(ug1, ube1), _pack_bn(ig1, ibe1),
                      _pack_bn(ug2, ube2), _pack_bn(ig2, ibe2))
    return score.reshape(B)


# speedup vs baseline: 2.8270x; 2.8270x over previous
import functools

import jax
import jax.numpy as jnp
from jax import lax
from jax.experimental import pallas as pl
from jax.experimental.pallas import tpu as pltpu
from jax.experimental.pallas import tpu_sc as plsc

B = 16384
NF = 13
V = 100000
Vp = 102400
D = 32
DIN = NF * D
H1, H2 = 128, 64
EPS_BN = 1e-5
EPS_NORM = 1e-12

VB = 4096
NVB = Vp // VB
PR = NF * Vp


def _pack_body(src, dst):
    x = src[0]
    dst[:, 0:D] = jnp.swapaxes(x, 0, 1)


def _pack_call(tabT):
    return pl.pallas_call(
        _pack_body,
        grid=(NF, NVB),
        in_specs=[pl.BlockSpec((1, D, VB), lambda f, vc: (f, 0, vc))],
        out_specs=pl.BlockSpec((VB, 128), lambda f, vc: (f * NVB + vc, 0)),
        out_shape=jax.ShapeDtypeStruct((PR, 128), jnp.float32),
    )(tabT)


NC, NS = 2, 16
NW = NC * NS
RW = B // NW
IPW = RW * NF
IPG = 128
G = IPW // IPG
NBUF = 4
NGRP = G // NBUF


def _sc_gather_body(tab, rows_hbm, out, rows_v, pad_bufs, sem_g, sem_o):
    wid = lax.axis_index("s") * NC + lax.axis_index("c")
    pltpu.sync_copy(rows_hbm.at[wid], rows_v)
    obase = wid * IPW

    def group(g, carry):
        gets = [
            pltpu.async_copy(tab.at[rows_v.at[g * NBUF + b]],
                             pad_bufs.at[b], sem_g)
            for b in range(NBUF)
        ]
        puts = []
        for b in range(NBUF):
            gets[b].wait()
            puts.append(pltpu.async_copy(
                pad_bufs.at[b, :, pl.ds(0, D)],
                out.at[pl.ds(obase + (g * NBUF + b) * IPG, IPG)],
                sem_o))
        for put in puts:
            put.wait()
        return carry

    lax.fori_loop(0, NGRP, group, 0)


@functools.cache
def _sc_gather_call():
    mesh = plsc.VectorSubcoreMesh(core_axis_name="c", subcore_axis_name="s",
                                  num_cores=NC, num_subcores=NS)
    return pl.kernel(
        _sc_gather_body,
        mesh=mesh,
        out_type=jax.ShapeDtypeStruct((B * NF, D), jnp.float32),
        scratch_types=[pltpu.VMEM((G, IPG), jnp.int32),
                       pltpu.VMEM((NBUF, IPG, 128), jnp.float32),
                       pltpu.SemaphoreType.DMA,
                       pltpu.SemaphoreType.DMA],
        compiler_params=pltpu.CompilerParams(use_tc_tiling_on_sc=False,
                                             needs_layout_passes=False),
    )


BLK = 1024
NB = B // BLK


def _mlp_body(uf, itf, uW1r, iW1r, uW2r, iW2r, uv1, iv1, uv2, iv2, out,
              h1u, h1i, h2u, h2i, s1u, s1i, s2u, s2i):
    p = pl.program_id(0)
    i = pl.program_id(1)
    towers = ((uf, uW1r, uW2r, uv1, uv2, h1u, h2u, s1u, s2u),
              (itf, iW1r, iW2r, iv1, iv2, h1i, h2i, s1i, s2i))

    @pl.when(p == 0)
    def _():
        for f_ref, W1r, _W2r, _v1, _v2, h1, _h2, s1, _s2 in towers:
            x = f_ref[...]
            h = jnp.dot(x, W1r[...], preferred_element_type=jnp.float32)
            h1[pl.ds(i * BLK, BLK), :] = h
            st = jnp.concatenate(
                [jnp.sum(h, axis=0, keepdims=True),
                 jnp.sum(h * h, axis=0, keepdims=True)], axis=0)

            @pl.when(i == 0)
            def _():
                s1[0:2, :] = st

            @pl.when(i > 0)
            def _():
                s1[0:2, :] = s1[0:2, :] + st

    @pl.when(p == 1)
    def _():
        for _f, _W1r, W2r, v1, _v2, h1, h2, s1, s2 in towers:
            mu = s1[0:1, :] * (1.0 / B)
            var = s1[1:2, :] * (1.0 / B) - mu * mu
            a = v1[0:1, :] * lax.rsqrt(var + EPS_BN)
            c = v1[1:2, :] - mu * a
            h = h1[pl.ds(i * BLK, BLK), :]
            t = jnp.tanh(h * a + c)
            h2blk = jnp.dot(t, W2r[...], preferred_element_type=jnp.float32)
            h2[pl.ds(i * BLK, BLK), :] = h2blk
            st = jnp.concatenate(
                [jnp.sum(h2blk, axis=0, keepdims=True),
                 jnp.sum(h2blk * h2blk, axis=0, keepdims=True)], axis=0)

            @pl.when(i == 0)
            def _():
                s2[0:2, :] = st

            @pl.when(i > 0)
            def _():
                s2[0:2, :] = s2[0:2, :] + st

    @pl.when(p == 2)
    def _():
        zs = []
        for _f, _W1r, _W2r, _v1, v2, _h1, h2, _s1, s2 in towers:
            mu = s2[0:1, :] * (1.0 / B)
            var = s2[1:2, :] * (1.0 / B) - mu * mu
            a = v2[0:1, :] * lax.rsqrt(var + EPS_BN)
            c = v2[1:2, :] - mu * a
            zs.append(jnp.tanh(h2[pl.ds(i * BLK, BLK), :] * a + c))
        zu, zi = zs
        nu = jnp.maximum(jnp.sqrt(jnp.sum(zu * zu, axis=1, keepdims=True)),
                         EPS_NORM)
        ni = jnp.maximum(jnp.sqrt(jnp.sum(zi * zi, axis=1, keepdims=True)),
                         EPS_NORM)
        score = jnp.sum(zu * zi, axis=1, keepdims=True) / (nu * ni)
        out[pl.ds(i * BLK, BLK), :] = score


def _mlp_call(u_feat, i_feat, uW1, iW1, uW2, iW2, uv1, iv1, uv2, iv2):
    feat_spec = pl.BlockSpec((BLK, DIN),
                             lambda p, i: (jnp.where(p == 0, i, NB - 1), 0))
    whole = lambda shape: pl.BlockSpec(shape, lambda p, i: (0, 0))
    return pl.pallas_call(
        _mlp_body,
        grid=(3, NB),
        in_specs=[feat_spec, feat_spec,
                  whole((DIN, H1)), whole((DIN, H1)),
                  whole((H1, H2)), whole((H1, H2)),
                  whole((8, H1)), whole((8, H1)),
                  whole((8, H2)), whole((8, H2))],
        out_specs=pl.BlockSpec((B, 1), lambda p, i: (0, 0)),
        out_shape=jax.ShapeDtypeStruct((B, 1), jnp.float32),
        scratch_shapes=[pltpu.VMEM((B, H1), jnp.float32),
                        pltpu.VMEM((B, H1), jnp.float32),
                        pltpu.VMEM((B, H2), jnp.float32),
                        pltpu.VMEM((B, H2), jnp.float32),
                        pltpu.VMEM((8, H1), jnp.float32),
                        pltpu.VMEM((8, H1), jnp.float32),
                        pltpu.VMEM((8, H2), jnp.float32),
                        pltpu.VMEM((8, H2), jnp.float32)],
    )(u_feat, i_feat, uW1, iW1, uW2, iW2, uv1, iv1, uv2, iv2)


def _pack_bn(g, be):
    v = jnp.stack([g, be])
    return jnp.concatenate([v, jnp.zeros((6, v.shape[1]), jnp.float32)], axis=0)


def kernel(user_inputs, item_inputs, user_tables, item_tables,
           uW1, ub1, ug1, ube1, uW2, ub2, ug2, ube2,
           iW1, ib1, ig1, ibe1, iW2, ib2, ig2, ibe2):
    offs = (jnp.arange(NF, dtype=jnp.int32) * Vp)[None, :]
    u_rows = (user_inputs.astype(jnp.int32) + offs).reshape(NW, G, IPG)
    i_rows = (item_inputs.astype(jnp.int32) + offs).reshape(NW, G, IPG)
    tabT_u = jnp.swapaxes(user_tables, 1, 2)
    tabT_i = jnp.swapaxes(item_tables, 1, 2)
    pu = _pack_call(tabT_u)
    u_flat = _sc_gather_call()(pu, u_rows)
    pi = _pack_call(tabT_i)
    i_flat = _sc_gather_call()(pi, i_rows)
    u_feat = u_flat.reshape(B, DIN)
    i_feat = i_flat.reshape(B, DIN)
    score = _mlp_call(u_feat, i_feat, uW1, iW1, uW2, iW2,
                      _pack_bn(ug1, ube1), _pack_bn(ig1, ibe1),
                      _pack_bn(ug2, ube2), _pack_bn(ig2, ibe2))
    return score.reshape(B)


# speedup vs baseline: 3.4808x; 1.2313x over previous
import functools

import jax
import jax.numpy as jnp
from jax import lax
from jax.experimental import pallas as pl
from jax.experimental.pallas import tpu as pltpu
from jax.experimental.pallas import tpu_sc as plsc

B = 16384
NF = 13
V = 100000
Vp = 102400
D = 32
DIN = NF * D
H1, H2 = 128, 64
EPS_BN = 1e-5
EPS_NORM = 1e-12

Q = Vp // 4
VB = 6400
NVB = Q // VB
PR = NF * Q


def _pack_body(u0, u1, u2, u3, i0, i1, i2, i3, uo_ref, io_ref):
    for srcs, dst in (((u0, u1, u2, u3), uo_ref), ((i0, i1, i2, i3), io_ref)):
        for k, s in enumerate(srcs):
            dst[:, k * D:(k + 1) * D] = jnp.swapaxes(s[0], 0, 1)


def _pack_call(tabT_u, tabT_i):
    def spec(k):
        return pl.BlockSpec((1, D, VB),
                            lambda f, vc, _k=k: (f, 0, _k * NVB + vc))
    in_specs = [spec(k) for k in range(4)] * 2
    out_spec = pl.BlockSpec((VB, 128), lambda f, vc: (f * NVB + vc, 0))
    return pl.pallas_call(
        _pack_body,
        grid=(NF, NVB),
        in_specs=in_specs,
        out_specs=[out_spec, out_spec],
        out_shape=[jax.ShapeDtypeStruct((PR, 128), jnp.float32),
                   jax.ShapeDtypeStruct((PR, 128), jnp.float32)],
    )(tabT_u, tabT_u, tabT_u, tabT_u, tabT_i, tabT_i, tabT_i, tabT_i)


NC, NS = 2, 16
NW = NC * NS
RW = B // NW
IPW = RW * NF
IPG = 128
G = IPW // IPG
NBUF = 4
NGRP = G // NBUF


def _sc_gather_body(u_tab, i_tab, u_rows, i_rows, u_out, i_out,
                    rows_v, pad_bufs, sem_g, sem_o):
    wid = lax.axis_index("s") * NC + lax.axis_index("c")
    for tab, rows_hbm, out in ((u_tab, u_rows, u_out),
                               (i_tab, i_rows, i_out)):
        pltpu.sync_copy(rows_hbm.at[wid], rows_v)
        obase = wid * IPW

        def group(g, carry):
            gets = [
                pltpu.async_copy(tab.at[rows_v.at[g * NBUF + b]],
                                 pad_bufs.at[b], sem_g)
                for b in range(NBUF)
            ]
            puts = []
            for b in range(NBUF):
                gets[b].wait()
                puts.append(pltpu.async_copy(
                    pad_bufs.at[b],
                    out.at[pl.ds(obase + (g * NBUF + b) * IPG, IPG)],
                    sem_o))
            for put in puts:
                put.wait()
            return carry

        lax.fori_loop(0, NGRP, group, 0)


@functools.cache
def _sc_gather_call():
    mesh = plsc.VectorSubcoreMesh(core_axis_name="c", subcore_axis_name="s",
                                  num_cores=NC, num_subcores=NS)
    return pl.kernel(
        _sc_gather_body,
        mesh=mesh,
        out_type=[jax.ShapeDtypeStruct((B * NF, D), jnp.float32),
                  jax.ShapeDtypeStruct((B * NF, D), jnp.float32)],
        scratch_types=[pltpu.VMEM((G, IPG), jnp.int32),
                       pltpu.VMEM((NBUF, IPG, D), jnp.float32),
                       pltpu.SemaphoreType.DMA,
                       pltpu.SemaphoreType.DMA],
        compiler_params=pltpu.CompilerParams(use_tc_tiling_on_sc=False,
                                             needs_layout_passes=False),
    )


BLK = 1024
NB = B // BLK


def _mlp_body(uf, itf, uW1r, iW1r, uW2r, iW2r, uv1, iv1, uv2, iv2, out,
              h1u, h1i, h2u, h2i, s1u, s1i, s2u, s2i):
    p = pl.program_id(0)
    i = pl.program_id(1)
    towers = ((uf, uW1r, uW2r, uv1, uv2, h1u, h2u, s1u, s2u),
              (itf, iW1r, iW2r, iv1, iv2, h1i, h2i, s1i, s2i))

    @pl.when(p == 0)
    def _():
        for f_ref, W1r, _W2r, _v1, _v2, h1, _h2, s1, _s2 in towers:
            x = f_ref[...]
            h = jnp.dot(x, W1r[...], preferred_element_type=jnp.float32)
            h1[pl.ds(i * BLK, BLK), :] = h
            st = jnp.concatenate(
                [jnp.sum(h, axis=0, keepdims=True),
                 jnp.sum(h * h, axis=0, keepdims=True)], axis=0)

            @pl.when(i == 0)
            def _():
                s1[0:2, :] = st

            @pl.when(i > 0)
            def _():
                s1[0:2, :] = s1[0:2, :] + st

    @pl.when(p == 1)
    def _():
        for _f, _W1r, W2r, v1, _v2, h1, h2, s1, s2 in towers:
            mu = s1[0:1, :] * (1.0 / B)
            var = s1[1:2, :] * (1.0 / B) - mu * mu
            a = v1[0:1, :] * lax.rsqrt(var + EPS_BN)
            c = v1[1:2, :] - mu * a
            h = h1[pl.ds(i * BLK, BLK), :]
            t = jnp.tanh(h * a + c)
            h2blk = jnp.dot(t, W2r[...], preferred_element_type=jnp.float32)
            h2[pl.ds(i * BLK, BLK), :] = h2blk
            st = jnp.concatenate(
                [jnp.sum(h2blk, axis=0, keepdims=True),
                 jnp.sum(h2blk * h2blk, axis=0, keepdims=True)], axis=0)

            @pl.when(i == 0)
            def _():
                s2[0:2, :] = st

            @pl.when(i > 0)
            def _():
                s2[0:2, :] = s2[0:2, :] + st

    @pl.when(p == 2)
    def _():
        zs = []
        for _f, _W1r, _W2r, _v1, v2, _h1, h2, _s1, s2 in towers:
            mu = s2[0:1, :] * (1.0 / B)
            var = s2[1:2, :] * (1.0 / B) - mu * mu
            a = v2[0:1, :] * lax.rsqrt(var + EPS_BN)
            c = v2[1:2, :] - mu * a
            zs.append(jnp.tanh(h2[pl.ds(i * BLK, BLK), :] * a + c))
        zu, zi = zs
        nu = jnp.maximum(jnp.sqrt(jnp.sum(zu * zu, axis=1, keepdims=True)),
                         EPS_NORM)
        ni = jnp.maximum(jnp.sqrt(jnp.sum(zi * zi, axis=1, keepdims=True)),
                         EPS_NORM)
        score = jnp.sum(zu * zi, axis=1, keepdims=True) / (nu * ni)
        out[pl.ds(i * BLK, BLK), :] = score


def _mlp_call(u_feat, i_feat, uW1, iW1, uW2, iW2, uv1, iv1, uv2, iv2):
    feat_spec = pl.BlockSpec((BLK, DIN),
                             lambda p, i: (jnp.where(p == 0, i, NB - 1), 0))
    whole = lambda shape: pl.BlockSpec(shape, lambda p, i: (0, 0))
    return pl.pallas_call(
        _mlp_body,
        grid=(3, NB),
        in_specs=[feat_spec, feat_spec,
                  whole((DIN, H1)), whole((DIN, H1)),
                  whole((H1, H2)), whole((H1, H2)),
                  whole((8, H1)), whole((8, H1)),
                  whole((8, H2)), whole((8, H2))],
        out_specs=pl.BlockSpec((B, 1), lambda p, i: (0, 0)),
        out_shape=jax.ShapeDtypeStruct((B, 1), jnp.float32),
        scratch_shapes=[pltpu.VMEM((B, H1), jnp.float32),
                        pltpu.VMEM((B, H1), jnp.float32),
                        pltpu.VMEM((B, H2), jnp.float32),
                        pltpu.VMEM((B, H2), jnp.float32),
                        pltpu.VMEM((8, H1), jnp.float32),
                        pltpu.VMEM((8, H1), jnp.float32),
                        pltpu.VMEM((8, H2), jnp.float32),
                        pltpu.VMEM((8, H2), jnp.float32)],
    )(u_feat, i_feat, uW1, iW1, uW2, iW2, uv1, iv1, uv2, iv2)


def _pack_bn(g, be):
    v = jnp.stack([g, be])
    return jnp.concatenate([v, jnp.zeros((6, v.shape[1]), jnp.float32)], axis=0)


def kernel(user_inputs, item_inputs, user_tables, item_tables,
           uW1, ub1, ug1, ube1, uW2, ub2, ug2, ube2,
           iW1, ib1, ig1, ibe1, iW2, ib2, ig2, ibe2):
    offs = (jnp.arange(NF, dtype=jnp.int32) * Q)[None, :]
    vu = user_inputs.astype(jnp.int32)
    vi = item_inputs.astype(jnp.int32)
    u_rows = (4 * (offs + vu % Q) + vu // Q).reshape(NW, G, IPG)
    i_rows = (4 * (offs + vi % Q) + vi // Q).reshape(NW, G, IPG)
    tabT_u = jnp.swapaxes(user_tables, 1, 2)
    tabT_i = jnp.swapaxes(item_tables, 1, 2)
    p4u, p4i = _pack_call(tabT_u, tabT_i)
    pu = p4u.reshape(NF * Vp, D)
    pi = p4i.reshape(NF * Vp, D)
    u_flat, i_flat = _sc_gather_call()(pu, pi, u_rows, i_rows)
    u_feat = u_flat.reshape(B, DIN)
    i_feat = i_flat.reshape(B, DIN)
    score = _mlp_call(u_feat, i_feat, uW1, iW1, uW2, iW2,
                      _pack_bn(ug1, ube1), _pack_bn(ig1, ibe1),
                      _pack_bn(ug2, ube2), _pack_bn(ig2, ibe2))
    return score.reshape(B)


# speedup vs baseline: 6.7880x; 1.9501x over previous
import functools

import jax
import jax.numpy as jnp
from jax import lax
from jax.experimental import pallas as pl
from jax.experimental.pallas import tpu as pltpu
from jax.experimental.pallas import tpu_sc as plsc

B = 16384
NF = 13
V = 100000
Vp = 102400
D = 32
DIN = NF * D
H1, H2 = 128, 64
EPS_BN = 1e-5
EPS_NORM = 1e-12

Q = Vp // 4
VB = 6400
NVB = Q // VB
PR = NF * Q


def _pack_body(u0, u1, u2, u3, i0, i1, i2, i3, uo_ref, io_ref):
    for srcs, dst in (((u0, u1, u2, u3), uo_ref), ((i0, i1, i2, i3), io_ref)):
        y = jnp.concatenate([s[0] for s in srcs], axis=0)
        dst[...] = jnp.swapaxes(y, 0, 1)


def _pack_call(tabT_u, tabT_i):
    def spec(k):
        return pl.BlockSpec((1, D, VB),
                            lambda f, vc, _k=k: (f, 0, _k * NVB + vc))
    in_specs = [spec(k) for k in range(4)] * 2
    out_spec = pl.BlockSpec((VB, 128), lambda f, vc: (f * NVB + vc, 0))
    return pl.pallas_call(
        _pack_body,
        grid=(NF, NVB),
        in_specs=in_specs,
        out_specs=[out_spec, out_spec],
        out_shape=[jax.ShapeDtypeStruct((PR, 128), jnp.float32),
                   jax.ShapeDtypeStruct((PR, 128), jnp.float32)],
    )(tabT_u, tabT_u, tabT_u, tabT_u, tabT_i, tabT_i, tabT_i, tabT_i)


NC, NS = 2, 16
NW = NC * NS
RW = B // NW
IPW = RW * NF
IPG = 128
G = IPW // IPG
NBUF = 4
NGRP = G // NBUF


def _sc_gather_body(u_tab, i_tab, u_rows, i_rows, u_out, i_out,
                    rows_v, pad_bufs, sem_g, sem_o):
    wid = lax.axis_index("s") * NC + lax.axis_index("c")
    for tab, rows_hbm, out in ((u_tab, u_rows, u_out),
                               (i_tab, i_rows, i_out)):
        pltpu.sync_copy(rows_hbm.at[wid], rows_v)
        obase = wid * IPW

        def group(g, carry):
            gets = [
                pltpu.async_copy(tab.at[rows_v.at[g * NBUF + b]],
                                 pad_bufs.at[b], sem_g)
                for b in range(NBUF)
            ]
            puts = []
            for b in range(NBUF):
                gets[b].wait()
                puts.append(pltpu.async_copy(
                    pad_bufs.at[b],
                    out.at[pl.ds(obase + (g * NBUF + b) * IPG, IPG)],
                    sem_o))
            for put in puts:
                put.wait()
            return carry

        lax.fori_loop(0, NGRP, group, 0)


@functools.cache
def _sc_gather_call():
    mesh = plsc.VectorSubcoreMesh(core_axis_name="c", subcore_axis_name="s",
                                  num_cores=NC, num_subcores=NS)
    return pl.kernel(
        _sc_gather_body,
        mesh=mesh,
        out_type=[jax.ShapeDtypeStruct((B * NF, D), jnp.float32),
                  jax.ShapeDtypeStruct((B * NF, D), jnp.float32)],
        scratch_types=[pltpu.VMEM((G, IPG), jnp.int32),
                       pltpu.VMEM((NBUF, IPG, D), jnp.float32),
                       pltpu.SemaphoreType.DMA,
                       pltpu.SemaphoreType.DMA],
        compiler_params=pltpu.CompilerParams(use_tc_tiling_on_sc=False,
                                             needs_layout_passes=False),
    )


BLK = 1024
NB = B // BLK


def _mlp_body(uf, itf, uW1r, iW1r, uW2r, iW2r, uv1, iv1, uv2, iv2, out,
              h1u, h1i, h2u, h2i, s1u, s1i, s2u, s2i):
    p = pl.program_id(0)
    i = pl.program_id(1)
    towers = ((uf, uW1r, uW2r, uv1, uv2, h1u, h2u, s1u, s2u),
              (itf, iW1r, iW2r, iv1, iv2, h1i, h2i, s1i, s2i))

    @pl.when(p == 0)
    def _():
        for f_ref, W1r, _W2r, _v1, _v2, h1, _h2, s1, _s2 in towers:
            x = f_ref[...]
            h = jnp.dot(x, W1r[...], preferred_element_type=jnp.float32)
            h1[pl.ds(i * BLK, BLK), :] = h
            st = jnp.concatenate(
                [jnp.sum(h, axis=0, keepdims=True),
                 jnp.sum(h * h, axis=0, keepdims=True)], axis=0)

            @pl.when(i == 0)
            def _():
                s1[0:2, :] = st

            @pl.when(i > 0)
            def _():
                s1[0:2, :] = s1[0:2, :] + st

    @pl.when(p == 1)
    def _():
        for _f, _W1r, W2r, v1, _v2, h1, h2, s1, s2 in towers:
            mu = s1[0:1, :] * (1.0 / B)
            var = s1[1:2, :] * (1.0 / B) - mu * mu
            a = v1[0:1, :] * lax.rsqrt(var + EPS_BN)
            c = v1[1:2, :] - mu * a
            h = h1[pl.ds(i * BLK, BLK), :]
            t = jnp.tanh(h * a + c)
            h2blk = jnp.dot(t, W2r[...], preferred_element_type=jnp.float32)
            h2[pl.ds(i * BLK, BLK), :] = h2blk
            st = jnp.concatenate(
                [jnp.sum(h2blk, axis=0, keepdims=True),
                 jnp.sum(h2blk * h2blk, axis=0, keepdims=True)], axis=0)

            @pl.when(i == 0)
            def _():
                s2[0:2, :] = st

            @pl.when(i > 0)
            def _():
                s2[0:2, :] = s2[0:2, :] + st

    @pl.when(p == 2)
    def _():
        zs = []
        for _f, _W1r, _W2r, _v1, v2, _h1, h2, _s1, s2 in towers:
            mu = s2[0:1, :] * (1.0 / B)
            var = s2[1:2, :] * (1.0 / B) - mu * mu
            a = v2[0:1, :] * lax.rsqrt(var + EPS_BN)
            c = v2[1:2, :] - mu * a
            zs.append(jnp.tanh(h2[pl.ds(i * BLK, BLK), :] * a + c))
        zu, zi = zs
        nu = jnp.maximum(jnp.sqrt(jnp.sum(zu * zu, axis=1, keepdims=True)),
                         EPS_NORM)
        ni = jnp.maximum(jnp.sqrt(jnp.sum(zi * zi, axis=1, keepdims=True)),
                         EPS_NORM)
        score = jnp.sum(zu * zi, axis=1, keepdims=True) / (nu * ni)
        out[pl.ds(i * BLK, BLK), :] = score


def _mlp_call(u_feat, i_feat, uW1, iW1, uW2, iW2, uv1, iv1, uv2, iv2):
    feat_spec = pl.BlockSpec((BLK, DIN),
                             lambda p, i: (jnp.where(p == 0, i, NB - 1), 0))
    whole = lambda shape: pl.BlockSpec(shape, lambda p, i: (0, 0))
    return pl.pallas_call(
        _mlp_body,
        grid=(3, NB),
        in_specs=[feat_spec, feat_spec,
                  whole((DIN, H1)), whole((DIN, H1)),
                  whole((H1, H2)), whole((H1, H2)),
                  whole((8, H1)), whole((8, H1)),
                  whole((8, H2)), whole((8, H2))],
        out_specs=pl.BlockSpec((B, 1), lambda p, i: (0, 0)),
        out_shape=jax.ShapeDtypeStruct((B, 1), jnp.float32),
        scratch_shapes=[pltpu.VMEM((B, H1), jnp.float32),
                        pltpu.VMEM((B, H1), jnp.float32),
                        pltpu.VMEM((B, H2), jnp.float32),
                        pltpu.VMEM((B, H2), jnp.float32),
                        pltpu.VMEM((8, H1), jnp.float32),
                        pltpu.VMEM((8, H1), jnp.float32),
                        pltpu.VMEM((8, H2), jnp.float32),
                        pltpu.VMEM((8, H2), jnp.float32)],
    )(u_feat, i_feat, uW1, iW1, uW2, iW2, uv1, iv1, uv2, iv2)


def _pack_bn(g, be):
    v = jnp.stack([g, be])
    return jnp.concatenate([v, jnp.zeros((6, v.shape[1]), jnp.float32)], axis=0)


def kernel(user_inputs, item_inputs, user_tables, item_tables,
           uW1, ub1, ug1, ube1, uW2, ub2, ug2, ube2,
           iW1, ib1, ig1, ibe1, iW2, ib2, ig2, ibe2):
    offs = (jnp.arange(NF, dtype=jnp.int32) * Q)[None, :]
    vu = user_inputs.astype(jnp.int32)
    vi = item_inputs.astype(jnp.int32)
    u_rows = (4 * (offs + vu % Q) + vu // Q).reshape(NW, G, IPG)
    i_rows = (4 * (offs + vi % Q) + vi // Q).reshape(NW, G, IPG)
    tabT_u = jnp.swapaxes(user_tables, 1, 2)
    tabT_i = jnp.swapaxes(item_tables, 1, 2)
    p4u, p4i = _pack_call(tabT_u, tabT_i)
    pu = p4u.reshape(NF * Vp, D)
    pi = p4i.reshape(NF * Vp, D)
    u_flat, i_flat = _sc_gather_call()(pu, pi, u_rows, i_rows)
    u_feat = u_flat.reshape(B, DIN)
    i_feat = i_flat.reshape(B, DIN)
    score = _mlp_call(u_feat, i_feat, uW1, iW1, uW2, iW2,
                      _pack_bn(ug1, ube1), _pack_bn(ig1, ibe1),
                      _pack_bn(ug2, ube2), _pack_bn(ig2, ibe2))
    return score.reshape(B)


# speedup vs baseline: 8.5959x; 1.2663x over previous
import functools

import jax
import jax.numpy as jnp
from jax import lax
from jax.experimental import pallas as pl
from jax.experimental.pallas import tpu as pltpu
from jax.experimental.pallas import tpu_sc as plsc

B = 16384
NF = 13
V = 100000
Vp = 102400
D = 32
DIN = NF * D
H1, H2 = 128, 64
EPS_BN = 1e-5
EPS_NORM = 1e-12

Q = Vp // 4
VB = 6400
NVB = Q // VB
PR = NF * Q


def _pack_body(u0, u1, u2, u3, i0, i1, i2, i3, uo_ref, io_ref):
    for srcs, dst in (((u0, u1, u2, u3), uo_ref), ((i0, i1, i2, i3), io_ref)):
        y = jnp.concatenate([s[0] for s in srcs], axis=0)
        dst[...] = jnp.swapaxes(y, 0, 1)


def _pack_call(tabT_u, tabT_i):
    def spec(k):
        return pl.BlockSpec((1, D, VB),
                            lambda f, vc, _k=k: (f, 0, _k * NVB + vc))
    in_specs = [spec(k) for k in range(4)] * 2
    out_spec = pl.BlockSpec((VB, 128), lambda f, vc: (f * NVB + vc, 0))
    return pl.pallas_call(
        _pack_body,
        grid=(NF, NVB),
        in_specs=in_specs,
        out_specs=[out_spec, out_spec],
        out_shape=[jax.ShapeDtypeStruct((PR, 128), jnp.float32),
                   jax.ShapeDtypeStruct((PR, 128), jnp.float32)],
    )(tabT_u, tabT_u, tabT_u, tabT_u, tabT_i, tabT_i, tabT_i, tabT_i)


NC, NS = 2, 16
NW = NC * NS
RW = B // NW
IPW = RW * NF
IPG = 128
G = IPW // IPG
NBUF = 4
NGRP = G // NBUF


def _sc_gather_body(u_tab, i_tab, u_rows, i_rows, u_out, i_out,
                    rows_v, pad_bufs, sem_g, sem_o):
    wid = lax.axis_index("s") * NC + lax.axis_index("c")
    for tab, rows_hbm, out in ((u_tab, u_rows, u_out),
                               (i_tab, i_rows, i_out)):
        pltpu.sync_copy(rows_hbm.at[wid], rows_v)

        def group(g, carry):
            gets = [
                pltpu.async_copy(tab.at[rows_v.at[g * NBUF + b]],
                                 pad_bufs.at[b], sem_g)
                for b in range(NBUF)
            ]
            puts = []
            for b in range(NBUF):
                gets[b].wait()
                j = g * NBUF + b
                ct = j >> 4
                lane = ((j >> 2) & 3) * D
                b0 = wid * RW + (j & 3) * IPG
                puts.append(pltpu.async_copy(
                    pad_bufs.at[b],
                    out.at[ct, pl.ds(b0, IPG), pl.ds(lane, D)],
                    sem_o))
            for put in puts:
                put.wait()
            return carry

        lax.fori_loop(0, NGRP, group, 0)


@functools.cache
def _sc_gather_call():
    mesh = plsc.VectorSubcoreMesh(core_axis_name="c", subcore_axis_name="s",
                                  num_cores=NC, num_subcores=NS)
    return pl.kernel(
        _sc_gather_body,
        mesh=mesh,
        out_type=[jax.ShapeDtypeStruct((4, B, 128), jnp.float32),
                  jax.ShapeDtypeStruct((4, B, 128), jnp.float32)],
        scratch_types=[pltpu.VMEM((G, IPG), jnp.int32),
                       pltpu.VMEM((NBUF, IPG, D), jnp.float32),
                       pltpu.SemaphoreType.DMA,
                       pltpu.SemaphoreType.DMA],
        compiler_params=pltpu.CompilerParams(use_tc_tiling_on_sc=False,
                                             needs_layout_passes=False),
    )


BLK = 1024
NB = B // BLK


def _mlp_body(uf0, uf1, uf2, uf3, if0, if1, if2, if3,
              uW1r, iW1r, uW2r, iW2r, uv1, iv1, uv2, iv2, out,
              h1u, h1i, h2u, h2i, s1u, s1i, s2u, s2i):
    p = pl.program_id(0)
    i = pl.program_id(1)
    towers = (((uf0, uf1, uf2, uf3), uW1r, uW2r, uv1, uv2, h1u, h2u, s1u, s2u),
              ((if0, if1, if2, if3), iW1r, iW2r, iv1, iv2, h1i, h2i, s1i, s2i))

    @pl.when(p == 0)
    def _():
        lane = lax.broadcasted_iota(jnp.int32, (BLK, 128), 1)
        for f_refs, W1r, _W2r, _v1, _v2, h1, _h2, s1, _s2 in towers:
            h = jnp.zeros((BLK, H1), jnp.float32)
            for ct in range(4):
                x = f_refs[ct][0]
                if ct == 3:
                    x = jnp.where(lane < D, x, 0.0)
                h = h + jnp.dot(x, W1r[ct],
                                preferred_element_type=jnp.float32)
            h1[pl.ds(i * BLK, BLK), :] = h
            st = jnp.concatenate(
                [jnp.sum(h, axis=0, keepdims=True),
                 jnp.sum(h * h, axis=0, keepdims=True)], axis=0)

            @pl.when(i == 0)
            def _():
                s1[0:2, :] = st

            @pl.when(i > 0)
            def _():
                s1[0:2, :] = s1[0:2, :] + st

    @pl.when(p == 1)
    def _():
        for _f, _W1r, W2r, v1, _v2, h1, h2, s1, s2 in towers:
            mu = s1[0:1, :] * (1.0 / B)
            var = s1[1:2, :] * (1.0 / B) - mu * mu
            a = v1[0:1, :] * lax.rsqrt(var + EPS_BN)
            c = v1[1:2, :] - mu * a
            h = h1[pl.ds(i * BLK, BLK), :]
            t = jnp.tanh(h * a + c)
            h2blk = jnp.dot(t, W2r[...], preferred_element_type=jnp.float32)
            h2[pl.ds(i * BLK, BLK), :] = h2blk
            st = jnp.concatenate(
                [jnp.sum(h2blk, axis=0, keepdims=True),
                 jnp.sum(h2blk * h2blk, axis=0, keepdims=True)], axis=0)

            @pl.when(i == 0)
            def _():
                s2[0:2, :] = st

            @pl.when(i > 0)
            def _():
                s2[0:2, :] = s2[0:2, :] + st

    @pl.when(p == 2)
    def _():
        zs = []
        for _f, _W1r, _W2r, _v1, v2, _h1, h2, _s1, s2 in towers:
            mu = s2[0:1, :] * (1.0 / B)
            var = s2[1:2, :] * (1.0 / B) - mu * mu
            a = v2[0:1, :] * lax.rsqrt(var + EPS_BN)
            c = v2[1:2, :] - mu * a
            zs.append(jnp.tanh(h2[pl.ds(i * BLK, BLK), :] * a + c))
        zu, zi = zs
        nu = jnp.maximum(jnp.sqrt(jnp.sum(zu * zu, axis=1, keepdims=True)),
                         EPS_NORM)
        ni = jnp.maximum(jnp.sqrt(jnp.sum(zi * zi, axis=1, keepdims=True)),
                         EPS_NORM)
        score = jnp.sum(zu * zi, axis=1, keepdims=True) / (nu * ni)
        out[pl.ds(i * BLK, BLK), :] = score


def _mlp_call(u4, i4, uW1p, iW1p, uW2, iW2, uv1, iv1, uv2, iv2):
    def fspec(ct):
        return pl.BlockSpec(
            (1, BLK, 128),
            lambda p, i, _ct=ct: (_ct, jnp.where(p == 0, i, NB - 1), 0))
    whole = lambda shape: pl.BlockSpec(shape,
                                       lambda p, i, _n=len(shape): (0,) * _n)
    return pl.pallas_call(
        _mlp_body,
        grid=(3, NB),
        in_specs=[fspec(ct) for ct in range(4)] * 2
                 + [whole((4, 128, H1)), whole((4, 128, H1)),
                    whole((H1, H2)), whole((H1, H2)),
                    whole((8, H1)), whole((8, H1)),
                    whole((8, H2)), whole((8, H2))],
        out_specs=pl.BlockSpec((B, 1), lambda p, i: (0, 0)),
        out_shape=jax.ShapeDtypeStruct((B, 1), jnp.float32),
        scratch_shapes=[pltpu.VMEM((B, H1), jnp.float32),
                        pltpu.VMEM((B, H1), jnp.float32),
                        pltpu.VMEM((B, H2), jnp.float32),
                        pltpu.VMEM((B, H2), jnp.float32),
                        pltpu.VMEM((8, H1), jnp.float32),
                        pltpu.VMEM((8, H1), jnp.float32),
                        pltpu.VMEM((8, H2), jnp.float32),
                        pltpu.VMEM((8, H2), jnp.float32)],
    )(u4, u4, u4, u4, i4, i4, i4, i4,
      uW1p, iW1p, uW2, iW2, uv1, iv1, uv2, iv2)


def _pack_bn(g, be):
    v = jnp.stack([g, be])
    return jnp.concatenate([v, jnp.zeros((6, v.shape[1]), jnp.float32)], axis=0)


def kernel(user_inputs, item_inputs, user_tables, item_tables,
           uW1, ub1, ug1, ube1, uW2, ub2, ug2, ube2,
           iW1, ib1, ig1, ibe1, iW2, ib2, ig2, ibe2):
    offs = (jnp.arange(NF, dtype=jnp.int32) * Q)[None, :]

    def rows_of(v):
        r = 4 * (offs + v % Q) + v // Q
        r4 = r.reshape(NW, 4, IPG, NF).transpose(0, 3, 1, 2)
        return r4.reshape(NW, G, IPG)

    u_rows = rows_of(user_inputs.astype(jnp.int32))
    i_rows = rows_of(item_inputs.astype(jnp.int32))
    tabT_u = jnp.swapaxes(user_tables, 1, 2)
    tabT_i = jnp.swapaxes(item_tables, 1, 2)
    p4u, p4i = _pack_call(tabT_u, tabT_i)
    pu = p4u.reshape(NF * Vp, D)
    pi = p4i.reshape(NF * Vp, D)
    u4, i4 = _sc_gather_call()(pu, pi, u_rows, i_rows)

    def padW1(W):
        return jnp.concatenate(
            [W, jnp.zeros((4 * H1 - DIN, H1), jnp.float32)]).reshape(4, H1, H1)

    score = _mlp_call(u4, i4, padW1(uW1), padW1(iW1), uW2, iW2,
                      _pack_bn(ug1, ube1), _pack_bn(ig1, ibe1),
                      _pack_bn(ug2, ube2), _pack_bn(ig2, ibe2))
    return score.reshape(B)
